# baseline (speedup 1.0000x reference)
import functools

import jax
import jax.numpy as jnp
from jax import lax
from jax.experimental import pallas as pl
from jax.experimental.pallas import tpu as pltpu
from jax.experimental.pallas import tpu_sc as plsc

BS = 32
P = 196
D = 32
K = 512
PS = 16
NROW = 14

NC = 2
NS = 16
NW = NC * NS
IMG = NROW * PS


def _assign_body(img_ref, cent_ref, out_ref):
    x = img_ref[0]
    c = cent_ref[...]
    dot = jnp.dot(x, c, preferred_element_type=jnp.float32,
                  precision=lax.Precision.HIGHEST)
    cn = jnp.sum(c * c, axis=0, keepdims=True)
    score = cn - 2.0 * dot
    m = jnp.max(score, axis=1, keepdims=True)
    ids = lax.broadcasted_iota(jnp.int32, (P, K), 1)
    a = jnp.min(jnp.where(score >= m, ids, K), axis=1)
    ap = jnp.concatenate([a, jnp.zeros((256 - P,), jnp.int32)])
    out_ref[...] = ap.reshape(1, 2, 128)


def _assignment(image, centroids):
    return pl.pallas_call(
        _assign_body,
        grid=(BS,),
        in_specs=[
            pl.BlockSpec((1, P, D), lambda b: (b, 0, 0)),
            pl.BlockSpec((D, K), lambda b: (0, 0)),
        ],
        out_specs=pl.BlockSpec((1, 2, 128), lambda b: (b, 0, 0)),
        out_shape=jax.ShapeDtypeStruct((BS, 2, 128), jnp.int32),
    )(image, centroids)


def _decode_body(in_ref, out_ref):
    out_ref[...] = in_ref[0, :, :IMG].astype(jnp.int32)[None]


def _decode(packed8):
    return pl.pallas_call(
        _decode_body,
        grid=(BS,),
        in_specs=[pl.BlockSpec((1, IMG, 256), lambda b: (b, 0, 0))],
        out_specs=pl.BlockSpec((1, IMG, IMG), lambda b: (b, 0, 0)),
        out_shape=jax.ShapeDtypeStruct((BS, IMG, IMG), jnp.int32),
    )(packed8)


@functools.cache
def _sc_gather_kernel():
    mesh = plsc.VectorSubcoreMesh(core_axis_name="c", subcore_axis_name="s")

    @functools.partial(
        pl.kernel,
        mesh=mesh,
        out_type=jax.ShapeDtypeStruct((BS, 112, 128), jnp.int32),
        compiler_params=pltpu.CompilerParams(needs_layout_passes=False),
        scratch_types=[
            pltpu.VMEM_SHARED((K // 2, 128), jnp.int32),
            pltpu.VMEM((K // 2, 128), jnp.int32),
            pltpu.VMEM((2, 128), jnp.int32),
            pltpu.VMEM((112, 128), jnp.int32),
            pltpu.SemaphoreType.DMA,
        ],
    )
    def _sc_gather(table_hbm, assign_hbm, out_hbm, tab_sh, tab_v, a_v, out_v,
                   sem):
        cid = lax.axis_index("c")
        sid = lax.axis_index("s")
        wid = sid * NC + cid

        with jax.named_scope("stage_table"):
            @pl.when(sid == 0)
            def _stage():
                pltpu.sync_copy(table_hbm, tab_sh)

            plsc.subcore_barrier()
            pltpu.sync_copy(tab_sh, tab_v)
            pltpu.sync_copy(assign_hbm.at[wid], a_v)

        lanes = lax.iota(jnp.int32, PS)
        kvec = lanes & 3

        def body(r, carry):
            for g in range(4):
                pvec = r * NROW + g * 4 + (lanes >> 2)
                a_lane = plsc.load_gather(a_v, [pvec >> 7, pvec & 127])
                trow = a_lane >> 1
                tbase = (a_lane & 1) * 64
                for i in range(PS):
                    words = plsc.load_gather(
                        tab_v, [trow, tbase + i * 4 + kvec])
                    out_v[8 * r + (4 * i + g) // 8,
                          pl.ds(64 * (i % 2) + 16 * g, PS)] = words
            return carry

        with jax.named_scope("assemble"):
            lax.fori_loop(0, NROW, body, 0)
        with jax.named_scope("writeback"):
            pltpu.sync_copy(out_v, out_hbm.at[wid])

    return _sc_gather


def kernel(image, centroids, cluster_labels):
    assign = _assignment(image, centroids)
    y4 = jnp.transpose(cluster_labels).reshape(K, 64, 4)
    table32 = (y4[..., 0] | (y4[..., 1] << 8)
               | (y4[..., 2] << 16) | (y4[..., 3] << 24))
    table32 = table32.reshape(K // 2, 128)
    out32 = _sc_gather_kernel()(table32, assign)
    b8 = lax.bitcast_convert_type(out32, jnp.int8)
    return _decode(b8.reshape(BS, IMG, 256))


# speedup vs baseline: 1.4091x; 1.4091x over previous
import functools

import numpy as np

import jax
import jax.numpy as jnp
from jax import lax
from jax.experimental import pallas as pl
from jax.experimental.pallas import tpu as pltpu
from jax.experimental.pallas import tpu_sc as plsc

BS = 32
P = 196
D = 32
K = 512
PS = 16
NROW = 14

NC = 2
NS = 16
NW = NC * NS
IMG = NROW * PS


def _assign_body(img_ref, cent_ref, out_ref):
    x = img_ref[0]
    c = cent_ref[...]
    dot = jnp.dot(x, c, preferred_element_type=jnp.float32,
                  precision=lax.Precision.HIGHEST)
    cn = jnp.sum(c * c, axis=0, keepdims=True)
    score = cn - 2.0 * dot
    m = jnp.max(score, axis=1, keepdims=True)
    ids = lax.broadcasted_iota(jnp.int32, (P, K), 1)
    a = jnp.min(jnp.where(score >= m, ids, K), axis=1)
    ap = jnp.concatenate([a, jnp.zeros((256 - P,), jnp.int32)])
    out_ref[...] = ap.reshape(1, 2, 128)


def _assignment(image, centroids):
    return pl.pallas_call(
        _assign_body,
        grid=(BS,),
        in_specs=[
            pl.BlockSpec((1, P, D), lambda b: (b, 0, 0)),
            pl.BlockSpec((D, K), lambda b: (0, 0)),
        ],
        out_specs=pl.BlockSpec((1, 2, 128), lambda b: (b, 0, 0)),
        out_shape=jax.ShapeDtypeStruct((BS, 2, 128), jnp.int32),
    )(image, centroids)


def _decode_body(in_ref, p_ref, out_ref):
    w = in_ref[0]
    acc = jnp.zeros((IMG, IMG), jnp.float32)
    for s in range(4):
        plane = ((w >> (8 * s)) & 255).astype(jnp.bfloat16)
        acc += jnp.dot(plane, p_ref[s], preferred_element_type=jnp.float32)
    out_ref[...] = acc.astype(jnp.int32)[None]


def _byte_perm():
    p = np.zeros((4, 64, IMG), np.float32)
    for s in range(4):
        for m in range(56):
            p[s, m, 4 * m + s] = 1.0
    return jnp.asarray(p, dtype=jnp.bfloat16)


def _decode(packed32):
    return pl.pallas_call(
        _decode_body,
        grid=(BS,),
        in_specs=[
            pl.BlockSpec((1, IMG, 64), lambda b: (b, 0, 0)),
            pl.BlockSpec((4, 64, IMG), lambda b: (0, 0, 0)),
        ],
        out_specs=pl.BlockSpec((1, IMG, IMG), lambda b: (b, 0, 0)),
        out_shape=jax.ShapeDtypeStruct((BS, IMG, IMG), jnp.int32),
    )(packed32, _byte_perm())


@functools.cache
def _sc_gather_kernel():
    mesh = plsc.VectorSubcoreMesh(core_axis_name="c", subcore_axis_name="s")

    @functools.partial(
        pl.kernel,
        mesh=mesh,
        out_type=jax.ShapeDtypeStruct((BS, IMG, 64), jnp.int32),
        compiler_params=pltpu.CompilerParams(needs_layout_passes=False),
        scratch_types=[
            pltpu.VMEM_SHARED((K // 2, 128), jnp.int32),
            pltpu.VMEM((K // 2, 128), jnp.int32),
            pltpu.VMEM((2, 128), jnp.int32),
            pltpu.VMEM((IMG, 64), jnp.int32),
            pltpu.SemaphoreType.DMA,
        ],
    )
    def _sc_gather(table_hbm, assign_hbm, out_hbm, tab_sh, tab_v, a_v, out_v,
                   sem):
        cid = lax.axis_index("c")
        sid = lax.axis_index("s")
        wid = sid * NC + cid

        with jax.named_scope("stage_table"):
            @pl.when(sid == 0)
            def _stage():
                pltpu.sync_copy(table_hbm, tab_sh)

            plsc.subcore_barrier()
            pltpu.sync_copy(tab_sh, tab_v)
            pltpu.sync_copy(assign_hbm.at[wid], a_v)

        lanes = lax.iota(jnp.int32, PS)
        kvec = lanes & 3

        def body(r, carry):
            for g in range(4):
                pvec = r * NROW + g * 4 + (lanes >> 2)
                a_lane = plsc.load_gather(a_v, [pvec >> 7, pvec & 127])
                trow = a_lane >> 1
                tbase = (a_lane & 1) * 64
                for i in range(PS):
                    words = plsc.load_gather(
                        tab_v, [trow, tbase + i * 4 + kvec])
                    out_v[r * PS + i, pl.ds(PS * g, PS)] = words
            return carry

        with jax.named_scope("assemble"):
            lax.fori_loop(0, NROW, body, 0)
        with jax.named_scope("writeback"):
            pltpu.sync_copy(out_v, out_hbm.at[wid])

    return _sc_gather


def kernel(image, centroids, cluster_labels):
    assign = _assignment(image, centroids)
    y4 = jnp.transpose(cluster_labels).reshape(K, 64, 4)
    table32 = (y4[..., 0] | (y4[..., 1] << 8)
               | (y4[..., 2] << 16) | (y4[..., 3] << 24))
    table32 = table32.reshape(K // 2, 128)
    out32 = _sc_gather_kernel()(table32, assign)
    return _decode(out32)


# speedup vs baseline: 1.5663x; 1.1115x over previous
import functools

import numpy as np

import jax
import jax.numpy as jnp
from jax import lax
from jax.experimental import pallas as pl
from jax.experimental.pallas import tpu as pltpu
from jax.experimental.pallas import tpu_sc as plsc

BS = 32
P = 196
D = 32
K = 512
PS = 16
NROW = 14

NC = 2
NS = 16
NW = NC * NS
IMG = NROW * PS


_AB = 4


def _assign_body(img_ref, cent_ref, out_ref):
    x = img_ref[...]
    x = jnp.concatenate([x, jnp.zeros((_AB, 256 - P, D), jnp.float32)], axis=1)
    x = x.reshape(_AB * 256, D)
    c = cent_ref[...]
    dot = jnp.dot(x, c, preferred_element_type=jnp.float32,
                  precision=lax.Precision.HIGHEST)
    cn = jnp.sum(c * c, axis=0, keepdims=True)
    score = cn - 2.0 * dot
    m = jnp.max(score, axis=1, keepdims=True)
    ids = lax.broadcasted_iota(jnp.int32, (_AB * 256, K), 1)
    a = jnp.min(jnp.where(score >= m, ids, K), axis=1,
                keepdims=True)
    out_ref[...] = jnp.transpose(a).reshape(_AB, 2, 128)


def _assignment(image, centroids):
    return pl.pallas_call(
        _assign_body,
        grid=(BS // _AB,),
        in_specs=[
            pl.BlockSpec((_AB, P, D), lambda b: (b, 0, 0)),
            pl.BlockSpec((D, K), lambda b: (0, 0)),
        ],
        out_specs=pl.BlockSpec((_AB, 2, 128), lambda b: (b, 0, 0)),
        out_shape=jax.ShapeDtypeStruct((BS, 2, 128), jnp.int32),
    )(image, centroids)


def _decode_body(in_ref, p_ref, out_ref):
    w = in_ref[0]
    acc = jnp.zeros((IMG, IMG), jnp.float32)
    for s in range(4):
        plane = ((w >> (8 * s)) & 255).astype(jnp.bfloat16)
        acc += jnp.dot(plane, p_ref[s], preferred_element_type=jnp.float32)
    out_ref[...] = acc.astype(jnp.int32)[None]


def _byte_perm():
    p = np.zeros((4, 64, IMG), np.float32)
    for s in range(4):
        for m in range(56):
            p[s, m, 4 * m + s] = 1.0
    return jnp.asarray(p, dtype=jnp.bfloat16)


def _decode(packed32):
    return pl.pallas_call(
        _decode_body,
        grid=(BS,),
        in_specs=[
            pl.BlockSpec((1, IMG, 64), lambda b: (b, 0, 0)),
            pl.BlockSpec((4, 64, IMG), lambda b: (0, 0, 0)),
        ],
        out_specs=pl.BlockSpec((1, IMG, IMG), lambda b: (b, 0, 0)),
        out_shape=jax.ShapeDtypeStruct((BS, IMG, IMG), jnp.int32),
    )(packed32, _byte_perm())


@functools.cache
def _sc_gather_kernel():
    mesh = plsc.VectorSubcoreMesh(core_axis_name="c", subcore_axis_name="s")

    @functools.partial(
        pl.kernel,
        mesh=mesh,
        out_type=jax.ShapeDtypeStruct((BS, IMG, 64), jnp.int32),
        compiler_params=pltpu.CompilerParams(needs_layout_passes=False),
        scratch_types=[
            pltpu.VMEM_SHARED((K // 2, 128), jnp.int32),
            pltpu.VMEM((K // 2, 128), jnp.int32),
            pltpu.VMEM((2, 128), jnp.int32),
            pltpu.VMEM((IMG, 64), jnp.int32),
            pltpu.SemaphoreType.DMA,
        ],
    )
    def _sc_gather(table_hbm, assign_hbm, out_hbm, tab_sh, tab_v, a_v, out_v,
                   sem):
        cid = lax.axis_index("c")
        sid = lax.axis_index("s")
        wid = sid * NC + cid

        with jax.named_scope("stage_table"):
            @pl.when(sid == 0)
            def _stage():
                pltpu.sync_copy(table_hbm, tab_sh)

            plsc.subcore_barrier()
            pltpu.sync_copy(tab_sh, tab_v)
            pltpu.sync_copy(assign_hbm.at[wid], a_v)

        lanes = lax.iota(jnp.int32, PS)
        kvec = lanes & 3

        def body(r, carry):
            for g in range(4):
                pvec = r * NROW + g * 4 + (lanes >> 2)
                a_lane = plsc.load_gather(a_v, [pvec >> 7, pvec & 127])
                trow = a_lane >> 1
                tbase = (a_lane & 1) * 64
                for i in range(PS):
                    words = plsc.load_gather(
                        tab_v, [trow, tbase + i * 4 + kvec])
                    out_v[r * PS + i, pl.ds(PS * g, PS)] = words
            return carry

        with jax.named_scope("assemble"):
            lax.fori_loop(0, NROW, body, 0)
        with jax.named_scope("writeback"):
            pltpu.sync_copy(out_v, out_hbm.at[wid])

    return _sc_gather


def kernel(image, centroids, cluster_labels):
    assign = _assignment(image, centroids)
    y4 = jnp.transpose(cluster_labels).reshape(K, 64, 4)
    table32 = (y4[..., 0] | (y4[..., 1] << 8)
               | (y4[..., 2] << 16) | (y4[..., 3] << 24))
    table32 = table32.reshape(K // 2, 128)
    out32 = _sc_gather_kernel()(table32, assign)
    return _decode(out32)


# speedup vs baseline: 1.6469x; 1.0515x over previous
import functools

import numpy as np

import jax
import jax.numpy as jnp
from jax import lax
from jax.experimental import pallas as pl
from jax.experimental.pallas import tpu as pltpu
from jax.experimental.pallas import tpu_sc as plsc

BS = 32
P = 196
D = 32
K = 512
PS = 16
NROW = 14

NC = 2
NS = 16
NW = NC * NS
IMG = NROW * PS


_AB = 4


def _assign_body(img_ref, cent_ref, out_ref):
    x = img_ref[...].reshape(_AB * P, D)
    c = cent_ref[...]
    dot = jnp.dot(x, c, preferred_element_type=jnp.float32,
                  precision=lax.Precision.HIGHEST)
    cn = jnp.sum(c * c, axis=0, keepdims=True)
    score = cn - 2.0 * dot
    m = jnp.max(score, axis=1, keepdims=True)
    ids = lax.broadcasted_iota(jnp.int32, (_AB * P, K), 1)
    a = jnp.min(jnp.where(score >= m, ids, K), axis=1,
                keepdims=True)
    a = jnp.concatenate([a.reshape(_AB, P, 1),
                         jnp.zeros((_AB, 256 - P, 1), jnp.int32)], axis=1)
    out_ref[...] = jnp.transpose(a.reshape(_AB * 256, 1)).reshape(_AB, 2, 128)


def _assignment(image, centroids):
    return pl.pallas_call(
        _assign_body,
        grid=(BS // _AB,),
        in_specs=[
            pl.BlockSpec((_AB, P, D), lambda b: (b, 0, 0)),
            pl.BlockSpec((D, K), lambda b: (0, 0)),
        ],
        out_specs=pl.BlockSpec((_AB, 2, 128), lambda b: (b, 0, 0)),
        out_shape=jax.ShapeDtypeStruct((BS, 2, 128), jnp.int32),
    )(image, centroids)


def _decode_body(in_ref, p_ref, out_ref):
    w = in_ref[0]
    acc = jnp.zeros((IMG, IMG), jnp.float32)
    for s in range(4):
        plane = ((w >> (8 * s)) & 255).astype(jnp.bfloat16)
        acc += jnp.dot(plane, p_ref[s], preferred_element_type=jnp.float32)
    out_ref[...] = acc.astype(jnp.int32)[None]


def _byte_perm():
    p = np.zeros((4, 64, IMG), np.float32)
    for s in range(4):
        for m in range(56):
            p[s, m, 4 * m + s] = 1.0
    return jnp.asarray(p, dtype=jnp.bfloat16)


def _decode(packed32):
    return pl.pallas_call(
        _decode_body,
        grid=(BS,),
        in_specs=[
            pl.BlockSpec((1, IMG, 64), lambda b: (b, 0, 0)),
            pl.BlockSpec((4, 64, IMG), lambda b: (0, 0, 0)),
        ],
        out_specs=pl.BlockSpec((1, IMG, IMG), lambda b: (b, 0, 0)),
        out_shape=jax.ShapeDtypeStruct((BS, IMG, IMG), jnp.int32),
    )(packed32, _byte_perm())


@functools.cache
def _sc_gather_kernel():
    mesh = plsc.VectorSubcoreMesh(core_axis_name="c", subcore_axis_name="s")

    @functools.partial(
        pl.kernel,
        mesh=mesh,
        out_type=jax.ShapeDtypeStruct((BS, IMG, 64), jnp.int32),
        compiler_params=pltpu.CompilerParams(needs_layout_passes=False),
        scratch_types=[
            pltpu.VMEM_SHARED((K // 2, 128), jnp.int32),
            pltpu.VMEM((K // 2, 128), jnp.int32),
            pltpu.VMEM((2, 128), jnp.int32),
            pltpu.VMEM((IMG, 64), jnp.int32),
            pltpu.SemaphoreType.DMA,
        ],
    )
    def _sc_gather(table_hbm, assign_hbm, out_hbm, tab_sh, tab_v, a_v, out_v,
                   sem):
        cid = lax.axis_index("c")
        sid = lax.axis_index("s")
        wid = sid * NC + cid

        with jax.named_scope("stage_table"):
            @pl.when(sid == 0)
            def _stage():
                pltpu.sync_copy(table_hbm, tab_sh)

            plsc.subcore_barrier()
            pltpu.sync_copy(tab_sh, tab_v)
            pltpu.sync_copy(assign_hbm.at[wid], a_v)

        lanes = lax.iota(jnp.int32, PS)
        kvec = lanes & 3

        def body(r, carry):
            for g in range(4):
                pvec = r * NROW + g * 4 + (lanes >> 2)
                a_lane = plsc.load_gather(a_v, [pvec >> 7, pvec & 127])
                trow = a_lane >> 1
                tbase = (a_lane & 1) * 64
                for i in range(PS):
                    words = plsc.load_gather(
                        tab_v, [trow, tbase + i * 4 + kvec])
                    out_v[r * PS + i, pl.ds(PS * g, PS)] = words
            return carry

        with jax.named_scope("assemble"):
            lax.fori_loop(0, NROW, body, 0)
        with jax.named_scope("writeback"):
            pltpu.sync_copy(out_v, out_hbm.at[wid])

    return _sc_gather


def kernel(image, centroids, cluster_labels):
    assign = _assignment(image, centroids)
    y4 = jnp.transpose(cluster_labels).reshape(K, 64, 4)
    table32 = (y4[..., 0] | (y4[..., 1] << 8)
               | (y4[..., 2] << 16) | (y4[..., 3] << 24))
    table32 = table32.reshape(K // 2, 128)
    out32 = _sc_gather_kernel()(table32, assign)
    return _decode(out32)
